# Initial kernel scaffold; baseline (speedup 1.0000x reference)
import jax
import jax.numpy as jnp
from jax.experimental import pallas as pl


def kernel(input, kernels, a, b, table, table_row_lengths):
    raise NotImplementedError("write your pallas kernel here")


# speedup vs baseline: 1.8314x; 1.8314x over previous
import functools

import jax
import jax.numpy as jnp
from jax import lax
from jax.experimental import pallas as pl
from jax.experimental.pallas import tpu as pltpu
from jax.experimental.pallas import tpu_sc as plsc

K = 32768
C = 96
KH = KW = 3
M = 9
TABLE_SIZE = 16
R = 2.5
B, H, W = 4, 224, 224
D = C + M // (KH * KW)
HO, WO = H - KH + 1, W - KW + 1
KFLAT = C * KH * KW

NC, NS = 2, 16
NW = NC * NS
RPW = K // NW
CH = 64
NCH = RPW // CH


def _vote_body(x_ref, a_ref, bias_ref, table_ref, len_ref,
               rows_out, idx_out, len16_out, hist):
    b = pl.program_id(0)

    @pl.when(b == 0)
    def _init():
        for t in range(TABLE_SIZE):
            hist[t] = jnp.int32(0)

    x2 = x_ref[0]
    y = jax.lax.dot_general(a_ref[...], x2, (((1,), (0,)), ((), ())),
                            preferred_element_type=jnp.float32)
    y3 = y.reshape(16, H, W)
    d = jnp.zeros((HO, WO), jnp.float32)
    for s in range(KH * KW):
        dh, dw = s // KW, s % KW
        d = d + y3[s, dh:dh + HO, dw:dw + WO]
    votes = jnp.floor((d + bias_ref[0, 0]) / jnp.float32(R))
    bins = jnp.abs(lax.rem(votes.astype(jnp.int32), jnp.int32(TABLE_SIZE)))
    for t in range(TABLE_SIZE):
        hist[t] = hist[t] + jnp.sum((bins == t).astype(jnp.int32))

    @pl.when(b == B - 1)
    def _final():
        bi = jnp.int32(0)
        bv = hist[0]
        for t in range(1, TABLE_SIZE):
            v = hist[t]
            take = v > bv
            bi = jnp.where(take, jnp.int32(t), bi)
            bv = jnp.where(take, v, bv)
        idx_out[0] = bi
        ln = jnp.int32(0)
        for t in range(TABLE_SIZE):
            ln = jnp.where(bi == t, len_ref[t], ln)
        len16_out[...] = jnp.full((1, 16), ln, jnp.int32)
        acc = jnp.zeros((1, K), jnp.int32)
        for t in range(TABLE_SIZE):
            acc = acc + jnp.where(bi == t, table_ref[t:t + 1, :], jnp.int32(0))
        rows_out[...] = acc


def _vote_call(x4, a_pad, bias, table, lengths):
    return pl.pallas_call(
        _vote_body,
        grid=(B,),
        in_specs=[
            pl.BlockSpec((1, C, H * W), lambda b: (b, 0, 0)),
            pl.BlockSpec((16, C), lambda b: (0, 0)),
            pl.BlockSpec(memory_space=pltpu.SMEM),
            pl.BlockSpec((TABLE_SIZE, K), lambda b: (0, 0)),
            pl.BlockSpec(memory_space=pltpu.SMEM),
        ],
        out_specs=[
            pl.BlockSpec((1, K), lambda b: (0, 0)),
            pl.BlockSpec(memory_space=pltpu.SMEM),
            pl.BlockSpec((1, 16), lambda b: (0, 0)),
        ],
        out_shape=[
            jax.ShapeDtypeStruct((1, K), jnp.int32),
            jax.ShapeDtypeStruct((1,), jnp.int32),
            jax.ShapeDtypeStruct((1, 16), jnp.int32),
        ],
        scratch_shapes=[pltpu.SMEM((TABLE_SIZE,), jnp.int32)],
    )(x4, a_pad, bias, table, lengths)


def _gather_body(kern_hbm, rows_hbm, len16_hbm, zeros_hbm, out_hbm,
                 idx_v, buf, zbuf, len_v, sem):
    cid = lax.axis_index("c")
    sid = lax.axis_index("s")
    wid = sid * NC + cid
    base = wid * RPW
    pltpu.sync_copy(rows_hbm.at[pl.ds(wid * NCH, NCH)], idx_v)
    pltpu.sync_copy(len16_hbm, len_v)
    pltpu.sync_copy(zeros_hbm, zbuf)
    ln = len_v[...][0]

    def chunk(j, carry):
        start = base + j * CH
        full = ln >= start + CH
        empty = ln <= start

        @pl.when(jnp.logical_not(empty))
        def _gather():
            pltpu.async_copy(kern_hbm.at[idx_v.at[j]], buf, sem).wait()

        @pl.when(full)
        def _write_full():
            pltpu.sync_copy(buf, out_hbm.at[pl.ds(start, CH)])

        @pl.when(empty)
        def _write_zero():
            pltpu.sync_copy(zbuf, out_hbm.at[pl.ds(start, CH)])

        @pl.when(jnp.logical_not(full) & jnp.logical_not(empty))
        def _boundary():
            nvalid = ln - start

            def zrow(r, c):
                for v in range(KFLAT // 16):
                    buf[r, pl.ds(v * 16, 16)] = jnp.zeros((16,), jnp.float32)
                return c

            lax.fori_loop(nvalid, CH, zrow, 0)
            pltpu.sync_copy(buf, out_hbm.at[pl.ds(start, CH)])

        return carry

    lax.fori_loop(0, NCH, chunk, 0)


def _gather_call(kern2, rows_sc, len16, zeros_blk):
    gk = pl.kernel(
        _gather_body,
        out_type=jax.ShapeDtypeStruct((K, KFLAT), jnp.float32),
        mesh=plsc.VectorSubcoreMesh(core_axis_name="c", subcore_axis_name="s",
                                    num_cores=NC, num_subcores=NS),
        scratch_types=[
            pltpu.VMEM((NCH, CH), jnp.int32),
            pltpu.VMEM((CH, KFLAT), jnp.float32),
            pltpu.VMEM((CH, KFLAT), jnp.float32),
            pltpu.VMEM((16,), jnp.int32),
            pltpu.SemaphoreType.DMA,
        ],
        compiler_params=pltpu.CompilerParams(use_tc_tiling_on_sc=False),
    )
    return gk(kern2, rows_sc, len16, zeros_blk)


def kernel(input, kernels, a, b, table, table_row_lengths):
    x4 = input.reshape(B, C, H * W)
    amat = a.reshape(D, KH * KW)
    a_pad = jnp.zeros((16, C), jnp.float32).at[:KH * KW].set(amat[:C].T)
    bias = (b + 0.5 * jnp.sum(amat[C])).reshape(1, 1).astype(jnp.float32)
    table_i = table.astype(jnp.int32)
    lengths_i = table_row_lengths.astype(jnp.int32)

    rows2d, idx1, len16 = _vote_call(x4, a_pad, bias, table_i, lengths_i)
    rows = rows2d.reshape(K)
    index = idx1.reshape(())

    kern2 = kernels.reshape(K, KFLAT)
    zeros_blk = jnp.zeros((CH, KFLAT), jnp.float32)
    rows_sc = rows2d.reshape(NW * NCH, CH)
    act = _gather_call(kern2, rows_sc, len16.reshape(16), zeros_blk)
    return act.reshape(K, C, KH, KW), index, rows


# speedup vs baseline: 2.1440x; 1.1707x over previous
import functools

import jax
import jax.numpy as jnp
from jax import lax
from jax.experimental import pallas as pl
from jax.experimental.pallas import tpu as pltpu
from jax.experimental.pallas import tpu_sc as plsc

K = 32768
C = 96
KH = KW = 3
M = 9
TABLE_SIZE = 16
R = 2.5
B, H, W = 4, 224, 224
D = C + M // (KH * KW)
HO, WO = H - KH + 1, W - KW + 1
KFLAT = C * KH * KW

NC, NS = 2, 16
NW = NC * NS
RPW = K // NW
CH = 64
NCH = RPW // CH


def _vote_body(x_ref, a_ref, bias_ref, table_ref, len_ref,
               rows_out, idx_out, len16_out, rowsg_out, hist):
    b = pl.program_id(0)

    @pl.when(b == 0)
    def _init():
        for t in range(TABLE_SIZE):
            hist[t] = jnp.int32(0)

    x2 = x_ref[0]
    y = jax.lax.dot_general(a_ref[...], x2, (((1,), (0,)), ((), ())),
                            preferred_element_type=jnp.float32)
    y3 = y.reshape(16, H, W)
    d = jnp.zeros((HO, WO), jnp.float32)
    for s in range(KH * KW):
        dh, dw = s // KW, s % KW
        d = d + y3[s, dh:dh + HO, dw:dw + WO]
    votes = jnp.floor((d + bias_ref[0, 0]) / jnp.float32(R))
    bins = jnp.abs(lax.rem(votes.astype(jnp.int32), jnp.int32(TABLE_SIZE)))
    for t in range(TABLE_SIZE):
        hist[t] = hist[t] + jnp.sum((bins == t).astype(jnp.int32))

    @pl.when(b == B - 1)
    def _final():
        bi = jnp.int32(0)
        bv = hist[0]
        for t in range(1, TABLE_SIZE):
            v = hist[t]
            take = v > bv
            bi = jnp.where(take, jnp.int32(t), bi)
            bv = jnp.where(take, v, bv)
        idx_out[0] = bi
        ln = jnp.int32(0)
        for t in range(TABLE_SIZE):
            ln = jnp.where(bi == t, len_ref[t], ln)
        len16_out[...] = jnp.full((1, 16), ln, jnp.int32)
        acc = jnp.zeros((1, K), jnp.int32)
        for t in range(TABLE_SIZE):
            acc = acc + jnp.where(bi == t, table_ref[t:t + 1, :], jnp.int32(0))
        rows_out[...] = acc
        iota_k = lax.broadcasted_iota(jnp.int32, (1, K), 1)
        rowsg_out[...] = jnp.where(iota_k < ln, acc, jnp.int32(K))


def _vote_call(x4, a_pad, bias, table, lengths):
    return pl.pallas_call(
        _vote_body,
        grid=(B,),
        in_specs=[
            pl.BlockSpec((1, C, H * W), lambda b: (b, 0, 0)),
            pl.BlockSpec((16, C), lambda b: (0, 0)),
            pl.BlockSpec(memory_space=pltpu.SMEM),
            pl.BlockSpec((TABLE_SIZE, K), lambda b: (0, 0)),
            pl.BlockSpec(memory_space=pltpu.SMEM),
        ],
        out_specs=[
            pl.BlockSpec((1, K), lambda b: (0, 0)),
            pl.BlockSpec(memory_space=pltpu.SMEM),
            pl.BlockSpec((1, 16), lambda b: (0, 0)),
            pl.BlockSpec((1, K), lambda b: (0, 0)),
        ],
        out_shape=[
            jax.ShapeDtypeStruct((1, K), jnp.int32),
            jax.ShapeDtypeStruct((1,), jnp.int32),
            jax.ShapeDtypeStruct((1, 16), jnp.int32),
            jax.ShapeDtypeStruct((1, K), jnp.int32),
        ],
        scratch_shapes=[pltpu.SMEM((TABLE_SIZE,), jnp.int32)],
    )(x4, a_pad, bias, table, lengths)


KPAD = KFLAT + 32


def _gather_body(kern_hbm, rows_hbm, len16_hbm, zeros_hbm, out_hbm,
                 idx_v, buf, zbuf, len_v, sem):
    cid = lax.axis_index("c")
    sid = lax.axis_index("s")
    wid = sid * NC + cid
    base = wid * RPW
    pltpu.sync_copy(rows_hbm.at[pl.ds(wid * NCH, NCH)], idx_v)
    pltpu.sync_copy(len16_hbm, len_v)
    pltpu.sync_copy(zeros_hbm, zbuf)
    ln = len_v[...][0]

    def chunk(j, carry):
        start = base + j * CH

        @pl.when(ln > start)
        def _gather():
            pltpu.async_copy(kern_hbm.at[idx_v.at[j]], buf, sem).wait()
            pltpu.sync_copy(buf, out_hbm.at[pl.ds(start, CH)])

        @pl.when(ln <= start)
        def _write_zero():
            pltpu.sync_copy(zbuf, out_hbm.at[pl.ds(start, CH)])

        return carry

    lax.fori_loop(0, NCH, chunk, 0)


def _gather_call(kern_pad, rows_sc, len16, zeros_blk):
    gk = pl.kernel(
        _gather_body,
        out_type=jax.ShapeDtypeStruct((K, KPAD), jnp.float32),
        mesh=plsc.VectorSubcoreMesh(core_axis_name="c", subcore_axis_name="s",
                                    num_cores=NC, num_subcores=NS),
        scratch_types=[
            pltpu.VMEM((NCH, CH), jnp.int32),
            pltpu.VMEM((CH, KPAD), jnp.float32),
            pltpu.VMEM((CH, KPAD), jnp.float32),
            pltpu.VMEM((16,), jnp.int32),
            pltpu.SemaphoreType.DMA,
        ],
    )
    return gk(kern_pad, rows_sc, len16, zeros_blk)


def kernel(input, kernels, a, b, table, table_row_lengths):
    x4 = input.reshape(B, C, H * W)
    amat = a.reshape(D, KH * KW)
    a_pad = jnp.zeros((16, C), jnp.float32).at[:KH * KW].set(amat[:C].T)
    bias = (b + 0.5 * jnp.sum(amat[C])).reshape(1, 1).astype(jnp.float32)
    table_i = table.astype(jnp.int32)
    lengths_i = table_row_lengths.astype(jnp.int32)

    rows2d, idx1, len16, rowsg = _vote_call(x4, a_pad, bias, table_i, lengths_i)
    rows = rows2d.reshape(K)
    index = idx1.reshape(())

    kern_pad = jnp.pad(kernels.reshape(K, KFLAT), ((0, 8), (0, KPAD - KFLAT)))
    zeros_blk = jnp.zeros((CH, KPAD), jnp.float32)
    rows_sc = rowsg.reshape(NW * NCH, CH)
    act_pad = _gather_call(kern_pad, rows_sc, len16.reshape(16), zeros_blk)
    act = act_pad[:, :KFLAT].reshape(K, C, KH, KW)
    return act, index, rows


# speedup vs baseline: 4.9047x; 2.2877x over previous
import functools

import jax
import jax.numpy as jnp
from jax import lax
from jax.experimental import pallas as pl
from jax.experimental.pallas import tpu as pltpu
from jax.experimental.pallas import tpu_sc as plsc

K = 32768
C = 96
KH = KW = 3
M = 9
TABLE_SIZE = 16
R = 2.5
B, H, W = 4, 224, 224
D = C + M // (KH * KW)
HO, WO = H - KH + 1, W - KW + 1
KFLAT = C * KH * KW

NC, NS = 2, 16
NW = NC * NS
PPW = KFLAT // NW
NG = K // 16


def _vote_body(x_ref, a_ref, bias_ref, table_ref, len_ref,
               rows_out, idx_out, len16_out, hist):
    b = pl.program_id(0)

    @pl.when(b == 0)
    def _init():
        for t in range(TABLE_SIZE):
            hist[t] = jnp.int32(0)

    x2 = x_ref[0]
    y = jax.lax.dot_general(a_ref[...], x2, (((1,), (0,)), ((), ())),
                            preferred_element_type=jnp.float32)
    y3 = y.reshape(16, H, W)
    d = jnp.zeros((HO, WO), jnp.float32)
    for s in range(KH * KW):
        dh, dw = s // KW, s % KW
        d = d + y3[s, dh:dh + HO, dw:dw + WO]
    votes = jnp.floor((d + bias_ref[0, 0]) / jnp.float32(R))
    bins = jnp.abs(lax.rem(votes.astype(jnp.int32), jnp.int32(TABLE_SIZE)))
    for t in range(TABLE_SIZE):
        hist[t] = hist[t] + jnp.sum((bins == t).astype(jnp.int32))

    @pl.when(b == B - 1)
    def _final():
        bi = jnp.int32(0)
        bv = hist[0]
        for t in range(1, TABLE_SIZE):
            v = hist[t]
            take = v > bv
            bi = jnp.where(take, jnp.int32(t), bi)
            bv = jnp.where(take, v, bv)
        idx_out[0] = bi
        ln = jnp.int32(0)
        for t in range(TABLE_SIZE):
            ln = jnp.where(bi == t, len_ref[t], ln)
        len16_out[...] = jnp.full((1, 16), ln, jnp.int32)
        acc = jnp.zeros((1, K), jnp.int32)
        for t in range(TABLE_SIZE):
            acc = acc + jnp.where(bi == t, table_ref[t:t + 1, :], jnp.int32(0))
        rows_out[...] = acc


def _vote_call(x4, a_pad, bias, table, lengths):
    return pl.pallas_call(
        _vote_body,
        grid=(B,),
        in_specs=[
            pl.BlockSpec((1, C, H * W), lambda b: (b, 0, 0)),
            pl.BlockSpec((16, C), lambda b: (0, 0)),
            pl.BlockSpec(memory_space=pltpu.SMEM),
            pl.BlockSpec((TABLE_SIZE, K), lambda b: (0, 0)),
            pl.BlockSpec(memory_space=pltpu.SMEM),
        ],
        out_specs=[
            pl.BlockSpec((1, K), lambda b: (0, 0)),
            pl.BlockSpec(memory_space=pltpu.SMEM),
            pl.BlockSpec((1, 16), lambda b: (0, 0)),
        ],
        out_shape=[
            jax.ShapeDtypeStruct((1, K), jnp.int32),
            jax.ShapeDtypeStruct((1,), jnp.int32),
            jax.ShapeDtypeStruct((1, 16), jnp.int32),
        ],
        scratch_shapes=[pltpu.SMEM((TABLE_SIZE,), jnp.int32)],
    )(x4, a_pad, bias, table, lengths)


def _pgather_body(bank_hbm, rows_hbm, len16_hbm, out_hbm,
                  idx_v, vin, vout, len_v, sem, wsem):
    cid = lax.axis_index("c")
    sid = lax.axis_index("s")
    wid = sid * NC + cid
    wbase = wid * PPW
    pltpu.sync_copy(rows_hbm, idx_v)
    pltpu.sync_copy(len16_hbm, len_v)
    ln = len_v[...][0]
    ngv = ln // jnp.int32(16)
    rem = ln - ngv * jnp.int32(16)
    iota = lax.iota(jnp.int32, 16)
    zeros16 = jnp.zeros((16,), jnp.float32)

    def plane(j, carry):
        p = wbase + j
        pltpu.sync_copy(bank_hbm.at[p], vin)

        @pl.when(j > 0)
        def _wait_prev():
            pltpu.make_async_copy(vout, out_hbm.at[p - 1], wsem).wait()

        def grp(g, c):
            idx16 = idx_v[pl.ds(g * 16, 16)]
            vout[pl.ds(g * 16, 16)] = plsc.load_gather(vin, [idx16])
            return c

        lax.fori_loop(0, ngv, grp, 0)

        @pl.when(ngv < NG)
        def _boundary():
            idx16 = idx_v[pl.ds(ngv * 16, 16)]
            vals = plsc.load_gather(vin, [idx16])
            vout[pl.ds(ngv * 16, 16)] = jnp.where(iota < rem, vals, zeros16)

        def zgrp(g, c):
            vout[pl.ds(g * 16, 16)] = zeros16
            return c

        lax.fori_loop(ngv + 1, NG, zgrp, 0)
        pltpu.async_copy(vout, out_hbm.at[p], wsem)
        return carry

    lax.fori_loop(0, PPW, plane, 0)
    pltpu.make_async_copy(vout, out_hbm.at[wbase + PPW - 1], wsem).wait()


def _pgather_call(bank_t, rows, len16):
    gk = pl.kernel(
        _pgather_body,
        out_type=jax.ShapeDtypeStruct((KFLAT, K), jnp.float32),
        mesh=plsc.VectorSubcoreMesh(core_axis_name="c", subcore_axis_name="s",
                                    num_cores=NC, num_subcores=NS),
        scratch_types=[
            pltpu.VMEM((K,), jnp.int32),
            pltpu.VMEM((K,), jnp.float32),
            pltpu.VMEM((K,), jnp.float32),
            pltpu.VMEM((16,), jnp.int32),
            pltpu.SemaphoreType.DMA,
            pltpu.SemaphoreType.DMA,
        ],
        compiler_params=pltpu.CompilerParams(needs_layout_passes=False),
    )
    return gk(bank_t, rows, len16)


def kernel(input, kernels, a, b, table, table_row_lengths):
    x4 = input.reshape(B, C, H * W)
    amat = a.reshape(D, KH * KW)
    a_pad = jnp.zeros((16, C), jnp.float32).at[:KH * KW].set(amat[:C].T)
    bias = (b + 0.5 * jnp.sum(amat[C])).reshape(1, 1).astype(jnp.float32)
    table_i = table.astype(jnp.int32)
    lengths_i = table_row_lengths.astype(jnp.int32)

    rows2d, idx1, len16 = _vote_call(x4, a_pad, bias, table_i, lengths_i)
    rows = rows2d.reshape(K)
    index = idx1.reshape(())

    bank_t = kernels.transpose(2, 3, 1, 0).reshape(KFLAT, K)
    out_t = _pgather_call(bank_t, rows, len16.reshape(16))
    act = out_t.reshape(KH, KW, C, K).transpose(3, 2, 0, 1)
    return act, index, rows


# speedup vs baseline: 9.9572x; 2.0301x over previous
import functools

import jax
import jax.numpy as jnp
from jax import lax
from jax.experimental import pallas as pl
from jax.experimental.pallas import tpu as pltpu
from jax.experimental.pallas import tpu_sc as plsc

K = 32768
C = 96
KH = KW = 3
M = 9
TABLE_SIZE = 16
R = 2.5
B, H, W = 4, 224, 224
D = C + M // (KH * KW)
HO, WO = H - KH + 1, W - KW + 1
KFLAT = C * KH * KW

NC, NS = 2, 16
NW = NC * NS
PPW = KFLAT // NW
NG = K // 16


def _vote_body(x_ref, a_ref, bias_ref, table_ref, len_ref,
               rows_out, idx_out, len16_out, hist):
    b = pl.program_id(0)

    @pl.when(b == 0)
    def _init():
        for t in range(TABLE_SIZE):
            hist[t] = jnp.int32(0)

    x2 = x_ref[0].reshape(C, H * W)
    y = jax.lax.dot_general(a_ref[...], x2, (((1,), (0,)), ((), ())),
                            preferred_element_type=jnp.float32)
    y3 = y.reshape(16, H, W)
    d = jnp.zeros((HO, WO), jnp.float32)
    for s in range(KH * KW):
        dh, dw = s // KW, s % KW
        d = d + y3[s, dh:dh + HO, dw:dw + WO]
    votes = jnp.floor((d + bias_ref[0, 0]) / jnp.float32(R))
    bins = jnp.abs(lax.rem(votes.astype(jnp.int32), jnp.int32(TABLE_SIZE)))
    for t in range(TABLE_SIZE):
        hist[t] = hist[t] + jnp.sum((bins == t).astype(jnp.int32))

    @pl.when(b == B - 1)
    def _final():
        bi = jnp.int32(0)
        bv = hist[0]
        for t in range(1, TABLE_SIZE):
            v = hist[t]
            take = v > bv
            bi = jnp.where(take, jnp.int32(t), bi)
            bv = jnp.where(take, v, bv)
        idx_out[0] = bi
        ln = jnp.int32(0)
        for t in range(TABLE_SIZE):
            ln = jnp.where(bi == t, len_ref[t], ln)
        len16_out[...] = jnp.full((1, 16), ln, jnp.int32)
        acc = jnp.zeros((1, K), jnp.int32)
        for t in range(TABLE_SIZE):
            acc = acc + jnp.where(bi == t, table_ref[t:t + 1, :], jnp.int32(0))
        rows_out[...] = acc


def _vote_call(x4, a_pad, bias, table, lengths):
    return pl.pallas_call(
        _vote_body,
        grid=(B,),
        in_specs=[
            pl.BlockSpec((1, C, H, W), lambda b: (b, 0, 0, 0)),
            pl.BlockSpec((16, C), lambda b: (0, 0)),
            pl.BlockSpec(memory_space=pltpu.SMEM),
            pl.BlockSpec((TABLE_SIZE, K), lambda b: (0, 0)),
            pl.BlockSpec(memory_space=pltpu.SMEM),
        ],
        out_specs=[
            pl.BlockSpec((1, K), lambda b: (0, 0)),
            pl.BlockSpec(memory_space=pltpu.SMEM),
            pl.BlockSpec((1, 16), lambda b: (0, 0)),
        ],
        out_shape=[
            jax.ShapeDtypeStruct((1, K), jnp.int32),
            jax.ShapeDtypeStruct((1,), jnp.int32),
            jax.ShapeDtypeStruct((1, 16), jnp.int32),
        ],
        scratch_shapes=[pltpu.SMEM((TABLE_SIZE,), jnp.int32)],
    )(x4, a_pad, bias, table, lengths)


def _pgather_body(bank_hbm, rows_hbm, len16_hbm, out_hbm,
                  idx_v, vin, vout, len_v, sem, wsem):
    cid = lax.axis_index("c")
    sid = lax.axis_index("s")
    wid = sid * NC + cid
    wbase = wid * PPW
    pltpu.sync_copy(rows_hbm, idx_v)
    pltpu.sync_copy(len16_hbm, len_v)
    ln = len_v[...][0]
    ngv = ln // jnp.int32(16)
    rem = ln - ngv * jnp.int32(16)
    iota = lax.iota(jnp.int32, 16)
    zeros16 = jnp.zeros((16,), jnp.float32)

    def plane(j, carry):
        p = wbase + j
        pltpu.sync_copy(bank_hbm.at[p], vin)

        @pl.when(j > 0)
        def _wait_prev():
            pltpu.make_async_copy(vout, out_hbm.at[p - 1], wsem).wait()

        def grp(g, c):
            idx16 = idx_v[pl.ds(g * 16, 16)]
            vout[pl.ds(g * 16, 16)] = plsc.load_gather(vin, [idx16])
            return c

        lax.fori_loop(0, ngv, grp, 0)

        @pl.when(ngv < NG)
        def _boundary():
            idx16 = idx_v[pl.ds(ngv * 16, 16)]
            vals = plsc.load_gather(vin, [idx16])
            vout[pl.ds(ngv * 16, 16)] = jnp.where(iota < rem, vals, zeros16)

        @pl.when(j == 0)
        def _zero_tail():
            def zgrp(g, c):
                vout[pl.ds(g * 16, 16)] = zeros16
                return c

            lax.fori_loop(ngv + 1, NG, zgrp, 0)
        pltpu.async_copy(vout, out_hbm.at[p], wsem)
        return carry

    lax.fori_loop(0, PPW, plane, 0)
    pltpu.make_async_copy(vout, out_hbm.at[wbase + PPW - 1], wsem).wait()


def _pgather_call(bank_t, rows, len16):
    gk = pl.kernel(
        _pgather_body,
        out_type=jax.ShapeDtypeStruct((KFLAT, K), jnp.float32),
        mesh=plsc.VectorSubcoreMesh(core_axis_name="c", subcore_axis_name="s",
                                    num_cores=NC, num_subcores=NS),
        scratch_types=[
            pltpu.VMEM((K,), jnp.int32),
            pltpu.VMEM((K,), jnp.float32),
            pltpu.VMEM((K,), jnp.float32),
            pltpu.VMEM((16,), jnp.int32),
            pltpu.SemaphoreType.DMA,
            pltpu.SemaphoreType.DMA,
        ],
        compiler_params=pltpu.CompilerParams(needs_layout_passes=False),
    )
    return gk(bank_t, rows, len16)


def kernel(input, kernels, a, b, table, table_row_lengths):
    x4 = input
    amat = a.reshape(D, KH * KW)
    a_pad = jnp.zeros((16, C), jnp.float32).at[:KH * KW].set(amat[:C].T)
    bias = (b + 0.5 * jnp.sum(amat[C])).reshape(1, 1).astype(jnp.float32)
    table_i = table.astype(jnp.int32)
    lengths_i = table_row_lengths.astype(jnp.int32)

    rows2d, idx1, len16 = _vote_call(x4, a_pad, bias, table_i, lengths_i)
    rows = rows2d.reshape(K)
    index = idx1.reshape(())

    bank_t = kernels.transpose(2, 3, 1, 0).reshape(KFLAT, K)
    out_t = _pgather_call(bank_t, rows, len16.reshape(16))
    act = out_t.reshape(KH, KW, C, K).transpose(3, 2, 0, 1)
    return act, index, rows


# speedup vs baseline: 12.0671x; 1.2119x over previous
import functools

import jax
import jax.numpy as jnp
from jax import lax
from jax.experimental import pallas as pl
from jax.experimental.pallas import tpu as pltpu
from jax.experimental.pallas import tpu_sc as plsc

K = 32768
C = 96
KH = KW = 3
M = 9
TABLE_SIZE = 16
R = 2.5
B, H, W = 4, 224, 224
D = C + M // (KH * KW)
HO, WO = H - KH + 1, W - KW + 1
KFLAT = C * KH * KW

NC, NS = 2, 16
NW = NC * NS
PPW = KFLAT // NW
NG = K // 16


def _vote_body(x_ref, a_ref, bias_ref, table_ref, len_ref,
               rows_out, idx_out, len16_out, hist):
    b = pl.program_id(0)

    @pl.when(b == 0)
    def _init():
        for t in range(TABLE_SIZE):
            hist[t] = jnp.int32(0)

    x2 = x_ref[0].reshape(C, H * W)
    y = jax.lax.dot_general(a_ref[...], x2, (((1,), (0,)), ((), ())),
                            preferred_element_type=jnp.float32)
    y3 = y.reshape(16, H, W)
    d = jnp.zeros((HO, WO), jnp.float32)
    for s in range(KH * KW):
        dh, dw = s // KW, s % KW
        d = d + y3[s, dh:dh + HO, dw:dw + WO]
    votes = jnp.floor((d + bias_ref[0, 0]) / jnp.float32(R))
    bins = jnp.abs(lax.rem(votes.astype(jnp.int32), jnp.int32(TABLE_SIZE)))
    for t in range(TABLE_SIZE):
        hist[t] = hist[t] + jnp.sum((bins == t).astype(jnp.int32))

    @pl.when(b == B - 1)
    def _final():
        bi = jnp.int32(0)
        bv = hist[0]
        for t in range(1, TABLE_SIZE):
            v = hist[t]
            take = v > bv
            bi = jnp.where(take, jnp.int32(t), bi)
            bv = jnp.where(take, v, bv)
        idx_out[0] = bi
        ln = jnp.int32(0)
        for t in range(TABLE_SIZE):
            ln = jnp.where(bi == t, len_ref[t], ln)
        len16_out[...] = jnp.full((1, 16), ln, jnp.int32)
        acc = jnp.zeros((1, K), jnp.int32)
        for t in range(TABLE_SIZE):
            acc = acc + jnp.where(bi == t, table_ref[t:t + 1, :], jnp.int32(0))
        rows_out[...] = acc


def _vote_call(x4, a_pad, bias, table, lengths):
    return pl.pallas_call(
        _vote_body,
        grid=(B,),
        in_specs=[
            pl.BlockSpec((1, C, H, W), lambda b: (b, 0, 0, 0)),
            pl.BlockSpec((16, C), lambda b: (0, 0)),
            pl.BlockSpec(memory_space=pltpu.SMEM),
            pl.BlockSpec((TABLE_SIZE, K), lambda b: (0, 0)),
            pl.BlockSpec(memory_space=pltpu.SMEM),
        ],
        out_specs=[
            pl.BlockSpec((1, K), lambda b: (0, 0)),
            pl.BlockSpec(memory_space=pltpu.SMEM),
            pl.BlockSpec((1, 16), lambda b: (0, 0)),
        ],
        out_shape=[
            jax.ShapeDtypeStruct((1, K), jnp.int32),
            jax.ShapeDtypeStruct((1,), jnp.int32),
            jax.ShapeDtypeStruct((1, 16), jnp.int32),
        ],
        scratch_shapes=[pltpu.SMEM((TABLE_SIZE,), jnp.int32)],
    )(x4, a_pad, bias, table, lengths)


NG2 = K // 32


def _pgather_body(bank_hbm, pk_hbm, len16_hbm, out_hbm,
                  idx_v, vin0, vin1, vout, len_v, rsem0, rsem1, wsem):
    cid = lax.axis_index("c")
    sid = lax.axis_index("s")
    wid = sid * NC + cid
    wbase = wid * PPW
    pltpu.sync_copy(pk_hbm, idx_v)
    pltpu.sync_copy(len16_hbm, len_v)
    ln = len_v[...][0]
    np2 = ln // jnp.int32(32)
    remv = ln - np2 * jnp.int32(32)
    iota = lax.iota(jnp.int32, 16)
    zeros16 = jnp.zeros((16,), jnp.float32)
    mask16 = jnp.int32(0xFFFF)

    bufs = (vin0, vin1)
    rsems = (rsem0, rsem1)
    pltpu.async_copy(bank_hbm.at[wbase], vin0, rsem0)

    for j in range(PPW):
        p = wbase + j
        vin = bufs[j % 2]
        if j + 1 < PPW:
            pltpu.async_copy(bank_hbm.at[p + 1], bufs[(j + 1) % 2],
                             rsems[(j + 1) % 2])
        pltpu.make_async_copy(bank_hbm.at[p], vin, rsems[j % 2]).wait()
        if j > 0:
            pltpu.make_async_copy(vout, out_hbm.at[p - 1], wsem).wait()

        def pair(g2, c, vin=vin):
            pk = idx_v[pl.ds(g2 * 16, 16)]
            lo = pk & mask16
            hi = lax.shift_right_logical(pk, 16)
            vout[pl.ds(g2 * 32, 16)] = plsc.load_gather(vin, [lo])
            vout[pl.ds(g2 * 32 + 16, 16)] = plsc.load_gather(vin, [hi])
            return c

        lax.fori_loop(0, np2, pair, 0)

        @pl.when(np2 < NG2)
        def _boundary(vin=vin):
            pk = idx_v[pl.ds(np2 * 16, 16)]
            lo = pk & mask16
            hi = lax.shift_right_logical(pk, 16)
            v0 = plsc.load_gather(vin, [lo])
            v1 = plsc.load_gather(vin, [hi])
            vout[pl.ds(np2 * 32, 16)] = jnp.where(iota < remv, v0, zeros16)
            vout[pl.ds(np2 * 32 + 16, 16)] = jnp.where(iota < remv - 16,
                                                       v1, zeros16)

        if j == 0:
            def zgrp(g, c):
                vout[pl.ds(g * 16, 16)] = zeros16
                return c

            lax.fori_loop(np2 * 2 + 2, NG, zgrp, 0)

        pltpu.async_copy(vout, out_hbm.at[p], wsem)

    pltpu.make_async_copy(vout, out_hbm.at[wbase + PPW - 1], wsem).wait()


def _pgather_call(bank_t, rows_pk, len16):
    gk = pl.kernel(
        _pgather_body,
        out_type=jax.ShapeDtypeStruct((KFLAT, K), jnp.float32),
        mesh=plsc.VectorSubcoreMesh(core_axis_name="c", subcore_axis_name="s",
                                    num_cores=NC, num_subcores=NS),
        scratch_types=[
            pltpu.VMEM((K // 2,), jnp.int32),
            pltpu.VMEM((K,), jnp.float32),
            pltpu.VMEM((K,), jnp.float32),
            pltpu.VMEM((K,), jnp.float32),
            pltpu.VMEM((16,), jnp.int32),
            pltpu.SemaphoreType.DMA,
            pltpu.SemaphoreType.DMA,
            pltpu.SemaphoreType.DMA,
        ],
        compiler_params=pltpu.CompilerParams(needs_layout_passes=False),
    )
    return gk(bank_t, rows_pk, len16)


def kernel(input, kernels, a, b, table, table_row_lengths):
    x4 = input
    amat = a.reshape(D, KH * KW)
    a_pad = jnp.zeros((16, C), jnp.float32).at[:KH * KW].set(amat[:C].T)
    bias = (b + 0.5 * jnp.sum(amat[C])).reshape(1, 1).astype(jnp.float32)
    table_i = table.astype(jnp.int32)
    lengths_i = table_row_lengths.astype(jnp.int32)

    rows2d, idx1, len16 = _vote_call(x4, a_pad, bias, table_i, lengths_i)
    rows = rows2d.reshape(K)
    index = idx1.reshape(())

    bank_t = kernels.transpose(2, 3, 1, 0).reshape(KFLAT, K)
    r2 = rows.reshape(K // 32, 2, 16)
    rows_pk = (r2[:, 0, :] | (r2[:, 1, :] << 16)).reshape(K // 2)
    out_t = _pgather_call(bank_t, rows_pk, len16.reshape(16))
    act = out_t.reshape(KH, KW, C, K).transpose(3, 2, 0, 1)
    return act, index, rows
